# Initial kernel scaffold; baseline (speedup 1.0000x reference)
import jax
import jax.numpy as jnp
from jax.experimental import pallas as pl


def kernel(indices, in_embed, out_embed):
    raise NotImplementedError("write your pallas kernel here")


# speedup vs baseline: 3.3815x; 3.3815x over previous
import functools

import jax
import jax.numpy as jnp
from jax import lax
from jax.experimental import pallas as pl
from jax.experimental.pallas import tpu as pltpu
from jax.experimental.pallas import tpu_sc as plsc

NC = 2
NS = 16
NW = NC * NS

VOCAB = 1000
D = 64
B_TOTAL = 1024 * 200
CHUNK = 128
N_CHUNKS = B_TOTAL // (NW * CHUNK)
PER_W = N_CHUNKS * CHUNK


def _body(idx_hbm, table_hbm, out_hbm, idx_v, rows0, rows1, sem0, sem1):
    wid = lax.axis_index("s") * NC + lax.axis_index("c")
    base = wid * PER_W

    pltpu.sync_copy(idx_hbm.at[wid], idx_v)

    pltpu.async_copy(table_hbm.at[idx_v.at[0]], rows0, sem0)
    pltpu.async_copy(table_hbm.at[idx_v.at[1]], rows1, sem1)

    @pl.loop(0, N_CHUNKS, step=2)
    def _(g):
        pltpu.make_async_copy(table_hbm.at[idx_v.at[g]], rows0, sem0).wait()
        pltpu.sync_copy(rows0, out_hbm.at[pl.ds(base + g * CHUNK, CHUNK)])

        @pl.when(g + 2 < N_CHUNKS)
        def _():
            pltpu.async_copy(table_hbm.at[idx_v.at[g + 2]], rows0, sem0)

        pltpu.make_async_copy(table_hbm.at[idx_v.at[g + 1]], rows1, sem1).wait()
        pltpu.sync_copy(rows1, out_hbm.at[pl.ds(base + (g + 1) * CHUNK, CHUNK)])

        @pl.when(g + 3 < N_CHUNKS)
        def _():
            pltpu.async_copy(table_hbm.at[idx_v.at[g + 3]], rows1, sem1)


@jax.jit
def _lookup(idx, in_embed):
    mesh = plsc.VectorSubcoreMesh(core_axis_name="c", subcore_axis_name="s")
    f = pl.kernel(
        _body,
        out_type=jax.ShapeDtypeStruct((B_TOTAL, D), jnp.float32),
        mesh=mesh,
        scratch_types=[
            pltpu.VMEM((N_CHUNKS, CHUNK), jnp.int32),
            pltpu.VMEM((CHUNK, D), jnp.float32),
            pltpu.VMEM((CHUNK, D), jnp.float32),
            pltpu.SemaphoreType.DMA,
            pltpu.SemaphoreType.DMA,
        ],
        compiler_params=pltpu.CompilerParams(use_tc_tiling_on_sc=False),
    )
    return f(idx, in_embed)


def kernel(indices, in_embed, out_embed):
    del out_embed
    bsz, hist = indices.shape
    idx = indices.astype(jnp.int32).reshape(NW, N_CHUNKS, CHUNK)
    out = _lookup(idx, in_embed)
    return out.reshape(bsz, hist, D)


# speedup vs baseline: 3.3896x; 1.0024x over previous
import functools

import jax
import jax.numpy as jnp
from jax import lax
from jax.experimental import pallas as pl
from jax.experimental.pallas import tpu as pltpu
from jax.experimental.pallas import tpu_sc as plsc

NC = 2
NS = 16
NW = NC * NS

VOCAB = 1000
D = 64
B_TOTAL = 1024 * 200
CHUNK = 128
N_CHUNKS = B_TOTAL // (NW * CHUNK)
PER_W = N_CHUNKS * CHUNK


RING = 10
LOOKAHEAD = 5


def _body(idx_hbm, table_hbm, out_hbm, idx_v, rows, gsem, wsem):
    wid = lax.axis_index("s") * NC + lax.axis_index("c")
    base = wid * PER_W

    pltpu.sync_copy(idx_hbm.at[wid], idx_v)

    for j in range(LOOKAHEAD):
        pltpu.async_copy(table_hbm.at[idx_v.at[j]], rows.at[j], gsem.at[j])

    @pl.loop(0, N_CHUNKS, step=RING)
    def _(g):
        for j in range(RING):
            c = g + j
            bf = (j + LOOKAHEAD) % RING

            def reuse(j=j, c=c, bf=bf):
                pltpu.make_async_copy(
                    rows.at[bf],
                    out_hbm.at[pl.ds(base + (c - LOOKAHEAD) * CHUNK, CHUNK)],
                    wsem.at[bf],
                ).wait()
                pltpu.async_copy(
                    table_hbm.at[idx_v.at[c + LOOKAHEAD]], rows.at[bf],
                    gsem.at[bf],
                )

            if j < LOOKAHEAD:
                @pl.when(g > 0)
                def _(reuse=reuse):
                    reuse()

                @pl.when(g == 0)
                def _(c=c, bf=bf):
                    pltpu.async_copy(
                        table_hbm.at[idx_v.at[c + LOOKAHEAD]], rows.at[bf],
                        gsem.at[bf],
                    )
            else:
                @pl.when(g < N_CHUNKS - RING)
                def _(reuse=reuse):
                    reuse()

                @pl.when(g == N_CHUNKS - RING)
                def _(c=c, bf=bf):
                    pltpu.make_async_copy(
                        rows.at[bf],
                        out_hbm.at[pl.ds(base + (c - LOOKAHEAD) * CHUNK, CHUNK)],
                        wsem.at[bf],
                    ).wait()

            pltpu.make_async_copy(
                table_hbm.at[idx_v.at[c]], rows.at[j], gsem.at[j]
            ).wait()
            pltpu.async_copy(
                rows.at[j], out_hbm.at[pl.ds(base + c * CHUNK, CHUNK)],
                wsem.at[j],
            )

    for j in range(RING - LOOKAHEAD, RING):
        c = N_CHUNKS - RING + j
        pltpu.make_async_copy(
            rows.at[j], out_hbm.at[pl.ds(base + c * CHUNK, CHUNK)], wsem.at[j]
        ).wait()


@jax.jit
def _lookup(idx, in_embed):
    mesh = plsc.VectorSubcoreMesh(core_axis_name="c", subcore_axis_name="s")
    f = pl.kernel(
        _body,
        out_type=jax.ShapeDtypeStruct((B_TOTAL, D), jnp.float32),
        mesh=mesh,
        scratch_types=[
            pltpu.VMEM((N_CHUNKS, CHUNK), jnp.int32),
            pltpu.VMEM((RING, CHUNK, D), jnp.float32),
            pltpu.SemaphoreType.DMA((RING,)),
            pltpu.SemaphoreType.DMA((RING,)),
        ],
        compiler_params=pltpu.CompilerParams(use_tc_tiling_on_sc=False),
    )
    return f(idx, in_embed)


def kernel(indices, in_embed, out_embed):
    del out_embed
    bsz, hist = indices.shape
    idx = indices.astype(jnp.int32).reshape(NW, N_CHUNKS, CHUNK)
    out = _lookup(idx, in_embed)
    return out.reshape(bsz, hist, D)


# speedup vs baseline: 3.3973x; 1.0023x over previous
import functools

import jax
import jax.numpy as jnp
from jax import lax
from jax.experimental import pallas as pl
from jax.experimental.pallas import tpu as pltpu
from jax.experimental.pallas import tpu_sc as plsc

NC = 2
NS = 16
NW = NC * NS

BATCH = 1024
HIST = 200
D = 64
HALF = HIST // 2
B_PER_W = BATCH // NW
RING = 8
LOOKAHEAD = 4


def _body(idx_hbm, table_hbm, out_hbm, idx_v, rows, gsem, wsem):
    wid = lax.axis_index("s") * NC + lax.axis_index("c")
    base = wid * B_PER_W

    pltpu.sync_copy(idx_hbm.at[wid], idx_v)

    def gathers(k, slot):
        pltpu.async_copy(
            table_hbm.at[idx_v.at[2 * k]],
            rows.at[slot].at[pl.ds(0, HALF)], gsem.at[slot])
        pltpu.async_copy(
            table_hbm.at[idx_v.at[2 * k + 1]],
            rows.at[slot].at[pl.ds(HALF, HALF)], gsem.at[slot])

    def wait_gathers(k, slot):
        pltpu.make_async_copy(
            table_hbm.at[idx_v.at[2 * k]],
            rows.at[slot].at[pl.ds(0, HALF)], gsem.at[slot]).wait()
        pltpu.make_async_copy(
            table_hbm.at[idx_v.at[2 * k + 1]],
            rows.at[slot].at[pl.ds(HALF, HALF)], gsem.at[slot]).wait()

    def write(k, slot):
        pltpu.async_copy(rows.at[slot], out_hbm.at[base + k], wsem.at[slot])

    def wait_write(k, slot):
        pltpu.make_async_copy(
            rows.at[slot], out_hbm.at[base + k], wsem.at[slot]).wait()

    for j in range(LOOKAHEAD):
        gathers(j, j)

    @pl.loop(0, B_PER_W, step=RING)
    def _(g):
        for j in range(RING):
            k = g + j
            bf = (j + LOOKAHEAD) % RING

            def reuse(k=k, bf=bf):
                wait_write(k - LOOKAHEAD, bf)
                gathers(k + LOOKAHEAD, bf)

            if j < LOOKAHEAD:
                @pl.when(g > 0)
                def _(reuse=reuse):
                    reuse()

                @pl.when(g == 0)
                def _(k=k, bf=bf):
                    gathers(k + LOOKAHEAD, bf)
            else:
                @pl.when(g < B_PER_W - RING)
                def _(reuse=reuse):
                    reuse()

                @pl.when(g == B_PER_W - RING)
                def _(k=k, bf=bf):
                    wait_write(k - LOOKAHEAD, bf)

            wait_gathers(k, j)
            write(k, j)

    for j in range(RING - LOOKAHEAD, RING):
        wait_write(B_PER_W - RING + j, j)


@jax.jit
def _lookup(idx, in_embed):
    mesh = plsc.VectorSubcoreMesh(core_axis_name="c", subcore_axis_name="s")
    f = pl.kernel(
        _body,
        out_type=jax.ShapeDtypeStruct((BATCH, HIST, D), jnp.float32),
        mesh=mesh,
        scratch_types=[
            pltpu.VMEM((2 * B_PER_W, HALF), jnp.int32),
            pltpu.VMEM((RING, HIST, D), jnp.float32),
            pltpu.SemaphoreType.DMA((RING,)),
            pltpu.SemaphoreType.DMA((RING,)),
        ],
        compiler_params=pltpu.CompilerParams(use_tc_tiling_on_sc=False),
    )
    return f(idx, in_embed)


def kernel(indices, in_embed, out_embed):
    del out_embed
    idx = indices.astype(jnp.int32).reshape(NW, 2 * B_PER_W, HALF)
    return _lookup(idx, in_embed)
